# Initial kernel scaffold; baseline (speedup 1.0000x reference)
import jax
import jax.numpy as jnp
from jax.experimental import pallas as pl


def kernel(x, edge_index, W1, b1, W2, b2, Wl, bl):
    raise NotImplementedError("write your pallas kernel here")


# speedup vs baseline: 11.9005x; 11.9005x over previous
import functools

import jax
import jax.numpy as jnp
from jax import lax
from jax.experimental import pallas as pl
from jax.experimental.pallas import tpu as pltpu
from jax.experimental.pallas import tpu_sc as plsc

N = 10000
E = 320000
NC = 2
NS = 16
NW = NC * NS
EW = E // NW
C = 80
NCHUNK = EW // C
RPS = 632
RPS_LAST = N - 15 * RPS

_MESH = plsc.VectorSubcoreMesh(core_axis_name="c", subcore_axis_name="s")


@functools.partial(
    pl.kernel,
    out_type=jax.ShapeDtypeStruct((NC, N, 128), jnp.float32),
    mesh=_MESH,
    scratch_types=[
        pltpu.VMEM((C,), jnp.int32),
        pltpu.VMEM((C, 128), jnp.float32),
        pltpu.VMEM_SHARED((N, 128), jnp.float32),
    ],
)
def _deg_kernel(dst_hbm, ones_hbm, zeros_hbm, out_hbm, dst_v, ones_v, acc_sh):
    cid = lax.axis_index("c")
    sid = lax.axis_index("s")
    wid = sid * NC + cid

    @pl.when(sid < NS - 1)
    def _():
        pltpu.sync_copy(zeros_hbm, acc_sh.at[pl.ds(sid * RPS, RPS)])

    @pl.when(sid == NS - 1)
    def _():
        pltpu.sync_copy(zeros_hbm.at[pl.ds(0, RPS_LAST)],
                        acc_sh.at[pl.ds(sid * RPS, RPS_LAST)])

    pltpu.sync_copy(ones_hbm, ones_v)
    plsc.subcore_barrier()

    def body(i, _):
        base = wid * EW + i * C
        pltpu.sync_copy(dst_hbm.at[pl.ds(base, C)], dst_v)
        pltpu.sync_copy(ones_v, acc_sh.at[dst_v], add=True)
        return 0

    lax.fori_loop(0, NCHUNK, body, 0)
    plsc.subcore_barrier()

    @pl.when(sid < NS - 1)
    def _():
        pltpu.sync_copy(acc_sh.at[pl.ds(sid * RPS, RPS)],
                        out_hbm.at[cid, pl.ds(sid * RPS, RPS)])

    @pl.when(sid == NS - 1)
    def _():
        pltpu.sync_copy(acc_sh.at[pl.ds(sid * RPS, RPS_LAST)],
                        out_hbm.at[cid, pl.ds(sid * RPS, RPS_LAST)])


@functools.partial(
    pl.kernel,
    out_type=jax.ShapeDtypeStruct((NC, N, 128), jnp.float32),
    mesh=_MESH,
    scratch_types=[
        pltpu.VMEM((C,), jnp.int32),
        pltpu.VMEM((C,), jnp.int32),
        pltpu.VMEM((C, 128), jnp.float32),
        pltpu.VMEM_SHARED((N, 128), jnp.float32),
        pltpu.SemaphoreType.DMA,
    ],
)
def _scatter_kernel(y_hbm, src_hbm, dst_hbm, zeros_hbm, out_hbm,
                    src_v, dst_v, rows_v, acc_sh, sem):
    cid = lax.axis_index("c")
    sid = lax.axis_index("s")
    wid = sid * NC + cid
    @pl.when(sid < NS - 1)
    def _():
        pltpu.sync_copy(zeros_hbm, acc_sh.at[pl.ds(sid * RPS, RPS)])

    @pl.when(sid == NS - 1)
    def _():
        pltpu.sync_copy(zeros_hbm.at[pl.ds(0, RPS_LAST)],
                        acc_sh.at[pl.ds(sid * RPS, RPS_LAST)])

    plsc.subcore_barrier()

    def body(i, _):
        base = wid * EW + i * C
        pltpu.sync_copy(src_hbm.at[pl.ds(base, C)], src_v)
        pltpu.sync_copy(dst_hbm.at[pl.ds(base, C)], dst_v)
        pltpu.async_copy(y_hbm.at[src_v], rows_v, sem).wait()
        pltpu.sync_copy(rows_v, acc_sh.at[dst_v], add=True)
        return 0

    lax.fori_loop(0, NCHUNK, body, 0)
    plsc.subcore_barrier()

    @pl.when(sid < NS - 1)
    def _():
        pltpu.sync_copy(acc_sh.at[pl.ds(sid * RPS, RPS)],
                        out_hbm.at[cid, pl.ds(sid * RPS, RPS)])

    @pl.when(sid == NS - 1)
    def _():
        pltpu.sync_copy(acc_sh.at[pl.ds(sid * RPS, RPS_LAST)],
                        out_hbm.at[cid, pl.ds(sid * RPS, RPS_LAST)])


_ROWS = 1000
_GRID = N // _ROWS


def _elu(v):
    return jnp.where(v > 0, v, jnp.exp(jnp.minimum(v, 0.0)) - 1.0)


def _t1_body(x_ref, w_ref, dinv_ref, y_ref):
    y_ref[...] = jnp.dot(x_ref[...], w_ref[...],
                         preferred_element_type=jnp.float32) * dinv_ref[...]


def _t1(x, W1, dinv):
    return pl.pallas_call(
        _t1_body,
        grid=(_GRID,),
        in_specs=[
            pl.BlockSpec((_ROWS, 128), lambda i: (i, 0)),
            pl.BlockSpec((128, 128), lambda i: (0, 0)),
            pl.BlockSpec((_ROWS, 1), lambda i: (i, 0)),
        ],
        out_specs=pl.BlockSpec((_ROWS, 128), lambda i: (i, 0)),
        out_shape=jax.ShapeDtypeStruct((N, 128), jnp.float32),
    )(x, W1, dinv)


def _t2_body(p0_ref, p1_ref, y_ref, dinv_ref, b_ref, w_ref, o_ref):
    agg = p0_ref[...] + p1_ref[...] + y_ref[...]
    h = _elu(agg * dinv_ref[...] + b_ref[...])
    o_ref[...] = jnp.dot(h, w_ref[...],
                         preferred_element_type=jnp.float32) * dinv_ref[...]


def _t2(p0, p1, y1, dinv, b1, W2):
    return pl.pallas_call(
        _t2_body,
        grid=(_GRID,),
        in_specs=[
            pl.BlockSpec((_ROWS, 128), lambda i: (i, 0)),
            pl.BlockSpec((_ROWS, 128), lambda i: (i, 0)),
            pl.BlockSpec((_ROWS, 128), lambda i: (i, 0)),
            pl.BlockSpec((_ROWS, 1), lambda i: (i, 0)),
            pl.BlockSpec((1, 128), lambda i: (0, 0)),
            pl.BlockSpec((128, 128), lambda i: (0, 0)),
        ],
        out_specs=pl.BlockSpec((_ROWS, 128), lambda i: (i, 0)),
        out_shape=jax.ShapeDtypeStruct((N, 128), jnp.float32),
    )(p0, p1, y1, dinv, b1, W2)


def _t3_body(p0_ref, p1_ref, y_ref, dinv_ref, b_ref, w_ref, bl_ref, o_ref):
    agg = p0_ref[...] + p1_ref[...] + y_ref[...]
    h = _elu(agg * dinv_ref[...] + b_ref[...])
    o_ref[...] = jnp.dot(h, w_ref[...],
                         preferred_element_type=jnp.float32) + bl_ref[...]


def _t3(p0, p1, y2, dinv, b2, Wl, bl):
    return pl.pallas_call(
        _t3_body,
        grid=(_GRID,),
        in_specs=[
            pl.BlockSpec((_ROWS, 128), lambda i: (i, 0)),
            pl.BlockSpec((_ROWS, 128), lambda i: (i, 0)),
            pl.BlockSpec((_ROWS, 128), lambda i: (i, 0)),
            pl.BlockSpec((_ROWS, 1), lambda i: (i, 0)),
            pl.BlockSpec((1, 128), lambda i: (0, 0)),
            pl.BlockSpec((128, 64), lambda i: (0, 0)),
            pl.BlockSpec((1, 64), lambda i: (0, 0)),
        ],
        out_specs=pl.BlockSpec((_ROWS, 64), lambda i: (i, 0)),
        out_shape=jax.ShapeDtypeStruct((N, 64), jnp.float32),
    )(p0, p1, y2, dinv, b2, Wl, bl)


def kernel(x, edge_index, W1, b1, W2, b2, Wl, bl):
    src = edge_index[0].astype(jnp.int32)
    dst = edge_index[1].astype(jnp.int32)
    ones = jnp.ones((C, 128), jnp.float32)
    zeros = jnp.zeros((RPS, 128), jnp.float32)

    degp = _deg_kernel(dst, ones, zeros)
    deg = degp[0, :, 0] + degp[1, :, 0] + 1.0
    dinv = lax.rsqrt(deg).reshape(N, 1)

    y1 = _t1(x, W1, dinv)
    p1 = _scatter_kernel(y1, src, dst, zeros)
    y2 = _t2(p1[0], p1[1], y1, dinv, b1.reshape(1, 128), W2)
    p2 = _scatter_kernel(y2, src, dst, zeros)
    out = _t3(p2[0], p2[1], y2, dinv, b2.reshape(1, 128), Wl,
              bl.reshape(1, 64))
    return out


# speedup vs baseline: 17.8313x; 1.4984x over previous
import functools

import jax
import jax.numpy as jnp
from jax import lax
from jax.experimental import pallas as pl
from jax.experimental.pallas import tpu as pltpu
from jax.experimental.pallas import tpu_sc as plsc

N = 10000
E = 320000
NC = 2
NS = 16
NW = NC * NS
EW = E // NW
C = 80
NCHUNK = EW // C
NPAIR = (NCHUNK - 1) // 2
RPS = 632
RPS_LAST = N - 15 * RPS

_MESH = plsc.VectorSubcoreMesh(core_axis_name="c", subcore_axis_name="s")


def _init_acc(sid, zeros_hbm, acc_sh):
    @pl.when(sid < NS - 1)
    def _():
        pltpu.sync_copy(zeros_hbm, acc_sh.at[pl.ds(sid * RPS, RPS)])

    @pl.when(sid == NS - 1)
    def _():
        pltpu.sync_copy(zeros_hbm.at[pl.ds(0, RPS_LAST)],
                        acc_sh.at[pl.ds(sid * RPS, RPS_LAST)])


def _copy_out(cid, sid, acc_sh, out_hbm):
    @pl.when(sid < NS - 1)
    def _():
        pltpu.sync_copy(acc_sh.at[pl.ds(sid * RPS, RPS)],
                        out_hbm.at[cid, pl.ds(sid * RPS, RPS)])

    @pl.when(sid == NS - 1)
    def _():
        pltpu.sync_copy(acc_sh.at[pl.ds(sid * RPS, RPS_LAST)],
                        out_hbm.at[cid, pl.ds(sid * RPS, RPS_LAST)])


@functools.partial(
    pl.kernel,
    out_type=jax.ShapeDtypeStruct((NC, N, 128), jnp.float32),
    mesh=_MESH,
    scratch_types=[
        pltpu.VMEM((C,), jnp.int32), pltpu.VMEM((C,), jnp.int32),
        pltpu.VMEM((C,), jnp.int32), pltpu.VMEM((C,), jnp.int32),
        pltpu.VMEM((C, 128), jnp.float32), pltpu.VMEM((C, 128), jnp.float32),
        pltpu.VMEM_SHARED((N, 128), jnp.float32),
        pltpu.SemaphoreType.DMA, pltpu.SemaphoreType.DMA,
        pltpu.SemaphoreType.DMA, pltpu.SemaphoreType.DMA,
    ],
)
def _scatter_kernel(y_hbm, src_hbm, dst_hbm, zeros_hbm, out_hbm,
                    src0, src1, dst0, dst1, rows0, rows1, acc_sh,
                    gsem0, gsem1, ssem0, ssem1):
    cid = lax.axis_index("c")
    sid = lax.axis_index("s")
    wid = sid * NC + cid
    ebase = wid * EW

    bufs = ((src0, dst0, rows0, gsem0, ssem0),
            (src1, dst1, rows1, gsem1, ssem1))

    def fire(i, b):
        sv, dv, rv, gs, _ = bufs[b]
        base = ebase + i * C
        pltpu.sync_copy(src_hbm.at[pl.ds(base, C)], sv)
        pltpu.sync_copy(dst_hbm.at[pl.ds(base, C)], dv)
        pltpu.async_copy(y_hbm.at[sv], rv, gs)

    def gather_wait(b):
        sv, _, rv, gs, _ = bufs[b]
        pltpu.make_async_copy(y_hbm.at[sv], rv, gs).wait()

    def scatter(b):
        _, dv, rv, _, ss = bufs[b]
        pltpu.async_copy(rv, acc_sh.at[dv], ss, add=True)

    def scatter_wait(b):
        _, dv, rv, _, ss = bufs[b]
        pltpu.make_async_copy(rv, acc_sh.at[dv], ss).wait()

    _init_acc(sid, zeros_hbm, acc_sh)
    plsc.subcore_barrier()

    fire(0, 0)
    fire(1, 1)

    def body(g, _):
        c0 = 2 * g
        gather_wait(0)
        scatter(0)
        gather_wait(1)
        scatter(1)
        scatter_wait(0)
        fire(c0 + 2, 0)

        @pl.when(c0 + 3 < NCHUNK)
        def _():
            scatter_wait(1)
            fire(c0 + 3, 1)

        return 0

    lax.fori_loop(0, NPAIR, body, 0)
    gather_wait(0)
    scatter(0)
    scatter_wait(1)
    scatter_wait(0)
    plsc.subcore_barrier()
    _copy_out(cid, sid, acc_sh, out_hbm)


@functools.partial(
    pl.kernel,
    out_type=jax.ShapeDtypeStruct((NC, N, 128), jnp.float32),
    mesh=_MESH,
    scratch_types=[
        pltpu.VMEM((C,), jnp.int32), pltpu.VMEM((C,), jnp.int32),
        pltpu.VMEM((C, 128), jnp.float32),
        pltpu.VMEM_SHARED((N, 128), jnp.float32),
        pltpu.SemaphoreType.DMA, pltpu.SemaphoreType.DMA,
    ],
)
def _deg_kernel(dst_hbm, ones_hbm, zeros_hbm, out_hbm,
                dst0, dst1, ones_v, acc_sh, ssem0, ssem1):
    cid = lax.axis_index("c")
    sid = lax.axis_index("s")
    wid = sid * NC + cid
    ebase = wid * EW

    bufs = ((dst0, ssem0), (dst1, ssem1))

    def load(i, b):
        dv, _ = bufs[b]
        pltpu.sync_copy(dst_hbm.at[pl.ds(ebase + i * C, C)], dv)

    def scatter(b):
        dv, ss = bufs[b]
        pltpu.async_copy(ones_v, acc_sh.at[dv], ss, add=True)

    def scatter_wait(b):
        dv, ss = bufs[b]
        pltpu.make_async_copy(ones_v, acc_sh.at[dv], ss).wait()

    _init_acc(sid, zeros_hbm, acc_sh)
    pltpu.sync_copy(ones_hbm, ones_v)
    plsc.subcore_barrier()

    load(0, 0)
    scatter(0)
    load(1, 1)
    scatter(1)

    def body(g, _):
        c0 = 2 * g
        scatter_wait(0)
        load(c0 + 2, 0)
        scatter(0)

        @pl.when(c0 + 3 < NCHUNK)
        def _():
            scatter_wait(1)
            load(c0 + 3, 1)
            scatter(1)

        return 0

    lax.fori_loop(0, NPAIR, body, 0)
    scatter_wait(0)
    scatter_wait(1)
    plsc.subcore_barrier()
    _copy_out(cid, sid, acc_sh, out_hbm)


_ROWS = 1000
_GRID = N // _ROWS


def _elu(v):
    return jnp.where(v > 0, v, jnp.exp(jnp.minimum(v, 0.0)) - 1.0)


def _t1_body(x_ref, w_ref, dinv_ref, y_ref):
    y_ref[...] = jnp.dot(x_ref[...], w_ref[...],
                         preferred_element_type=jnp.float32) * dinv_ref[...]


def _t1(x, W1, dinv):
    return pl.pallas_call(
        _t1_body,
        grid=(_GRID,),
        in_specs=[
            pl.BlockSpec((_ROWS, 128), lambda i: (i, 0)),
            pl.BlockSpec((128, 128), lambda i: (0, 0)),
            pl.BlockSpec((_ROWS, 1), lambda i: (i, 0)),
        ],
        out_specs=pl.BlockSpec((_ROWS, 128), lambda i: (i, 0)),
        out_shape=jax.ShapeDtypeStruct((N, 128), jnp.float32),
    )(x, W1, dinv)


def _t2_body(p0_ref, p1_ref, y_ref, dinv_ref, b_ref, w_ref, o_ref):
    agg = p0_ref[...] + p1_ref[...] + y_ref[...]
    h = _elu(agg * dinv_ref[...] + b_ref[...])
    o_ref[...] = jnp.dot(h, w_ref[...],
                         preferred_element_type=jnp.float32) * dinv_ref[...]


def _t2(p0, p1, y1, dinv, b1, W2):
    return pl.pallas_call(
        _t2_body,
        grid=(_GRID,),
        in_specs=[
            pl.BlockSpec((_ROWS, 128), lambda i: (i, 0)),
            pl.BlockSpec((_ROWS, 128), lambda i: (i, 0)),
            pl.BlockSpec((_ROWS, 128), lambda i: (i, 0)),
            pl.BlockSpec((_ROWS, 1), lambda i: (i, 0)),
            pl.BlockSpec((1, 128), lambda i: (0, 0)),
            pl.BlockSpec((128, 128), lambda i: (0, 0)),
        ],
        out_specs=pl.BlockSpec((_ROWS, 128), lambda i: (i, 0)),
        out_shape=jax.ShapeDtypeStruct((N, 128), jnp.float32),
    )(p0, p1, y1, dinv, b1, W2)


def _t3_body(p0_ref, p1_ref, y_ref, dinv_ref, b_ref, w_ref, bl_ref, o_ref):
    agg = p0_ref[...] + p1_ref[...] + y_ref[...]
    h = _elu(agg * dinv_ref[...] + b_ref[...])
    o_ref[...] = jnp.dot(h, w_ref[...],
                         preferred_element_type=jnp.float32) + bl_ref[...]


def _t3(p0, p1, y2, dinv, b2, Wl, bl):
    return pl.pallas_call(
        _t3_body,
        grid=(_GRID,),
        in_specs=[
            pl.BlockSpec((_ROWS, 128), lambda i: (i, 0)),
            pl.BlockSpec((_ROWS, 128), lambda i: (i, 0)),
            pl.BlockSpec((_ROWS, 128), lambda i: (i, 0)),
            pl.BlockSpec((_ROWS, 1), lambda i: (i, 0)),
            pl.BlockSpec((1, 128), lambda i: (0, 0)),
            pl.BlockSpec((128, 64), lambda i: (0, 0)),
            pl.BlockSpec((1, 64), lambda i: (0, 0)),
        ],
        out_specs=pl.BlockSpec((_ROWS, 64), lambda i: (i, 0)),
        out_shape=jax.ShapeDtypeStruct((N, 64), jnp.float32),
    )(p0, p1, y2, dinv, b2, Wl, bl)


def kernel(x, edge_index, W1, b1, W2, b2, Wl, bl):
    src = edge_index[0].astype(jnp.int32)
    dst = edge_index[1].astype(jnp.int32)
    ones = jnp.ones((C, 128), jnp.float32)
    zeros = jnp.zeros((RPS, 128), jnp.float32)

    degp = _deg_kernel(dst, ones, zeros)
    deg = degp[0, :, 0] + degp[1, :, 0] + 1.0
    dinv = lax.rsqrt(deg).reshape(N, 1)

    y1 = _t1(x, W1, dinv)
    p1 = _scatter_kernel(y1, src, dst, zeros)
    y2 = _t2(p1[0], p1[1], y1, dinv, b1.reshape(1, 128), W2)
    p2 = _scatter_kernel(y2, src, dst, zeros)
    out = _t3(p2[0], p2[1], y2, dinv, b2.reshape(1, 128), Wl,
              bl.reshape(1, 64))
    return out
